# baseline (speedup 1.0000x reference)
import functools

import jax
import jax.numpy as jnp
from jax import lax
from jax.experimental import pallas as pl
from jax.experimental.pallas import tpu as pltpu
from jax.experimental.pallas import tpu_sc as plsc

EMBED_DIM = 128
BATCH = 16384

_info = plsc.get_sparse_core_info()
NUM_CORES = _info.num_cores
NUM_SUBCORES = _info.num_subcores
NUM_WORKERS = NUM_CORES * NUM_SUBCORES
B_PER_W = BATCH // NUM_WORKERS
CHUNK = 128
NCHUNK = B_PER_W // CHUNK
NFEAT = 3


@functools.partial(
    pl.kernel,
    mesh=plsc.VectorSubcoreMesh(core_axis_name="c", subcore_axis_name="s"),
    out_type=[jax.ShapeDtypeStruct((BATCH, EMBED_DIM), jnp.float32)] * NFEAT,
    scratch_types=[
        pltpu.VMEM((NFEAT * B_PER_W,), jnp.int32),
        pltpu.VMEM((4, CHUNK, EMBED_DIM), jnp.float32),
    ] + [pltpu.SemaphoreType.DMA] * 8,
)
def _lookup3(idx_u, idx_i, idx_c, tab_u, tab_i, tab_c,
             out_u, out_i, out_c,
             idx_v, rows_v, *sems):
    wid = lax.axis_index("s") * NUM_CORES + lax.axis_index("c")
    base = wid * B_PER_W

    idx_hbm = [idx_u, idx_i, idx_c]
    tabs = [tab_u, tab_i, tab_c]
    outs = [out_u, out_i, out_c]
    NBUF = 4
    DRAIN_LAG = 2
    gsems = sems[:NBUF]
    ssems = sems[NBUF:]

    for f in range(NFEAT):
        pltpu.sync_copy(idx_hbm[f].at[pl.ds(base, B_PER_W)],
                        idx_v.at[pl.ds(f * B_PER_W, B_PER_W)])

    chunks = [(f, j) for f in range(NFEAT) for j in range(NCHUNK)]
    n = len(chunks)

    def gather_start(k, b):
        f, j = chunks[k]
        return pltpu.async_copy(
            tabs[f].at[idx_v.at[pl.ds(f * B_PER_W + j * CHUNK, CHUNK)]],
            rows_v.at[b], gsems[b])

    def store_start(k, b):
        f, j = chunks[k]
        return pltpu.async_copy(rows_v.at[b],
                                outs[f].at[pl.ds(base + j * CHUNK, CHUNK)],
                                ssems[b])

    g = [None] * NBUF
    s = [None] * NBUF
    for k in range(n + DRAIN_LAG):
        if k < n:
            b = k % NBUF
            if s[b] is not None:
                s[b].wait()
            g[b] = gather_start(k, b)
        d = k - DRAIN_LAG
        if d >= 0:
            bb = d % NBUF
            g[bb].wait()
            s[bb] = store_start(d, bb)
    for b in range(NBUF):
        if s[b] is not None:
            s[b].wait()


def kernel(user_id, item_id, category, table_user_id, table_item_id,
           table_category):
    out = _lookup3(user_id, item_id, category,
                   table_user_id, table_item_id, table_category)
    return tuple(out)
